# baseline (speedup 1.0000x reference)
import functools

import jax
import jax.numpy as jnp
from jax import lax
from jax.experimental import pallas as pl
from jax.experimental.pallas import tpu as pltpu
from jax.experimental.pallas import tpu_sc as plsc

_VOCAB = 1000000
_CLS_ID = _VOCAB
_SEP_ID = _VOCAB + 1

_Q = 200
_D = 26
_L = 50
_HEAD = _Q + 2
_N = _HEAD + _D * (_L + 1)
_NW = 32
_CHUNK = 48


def _body(query_hbm, domains_hbm, vocab_hbm, ids_out, seg_out,
          q_v, dom_v, idx_v, seg_v, rows_v,
          sem_q, sem_d, sem_g, sem_s, sem_i):
    wid = lax.axis_index("s") * 2 + lax.axis_index("c")
    base = pl.multiple_of(jnp.minimum(wid * _CHUNK, _N - _CHUNK), 8)

    cp_q = pltpu.async_copy(query_hbm, q_v, sem_q)
    cp_d = pltpu.async_copy(domains_hbm, dom_v, sem_d)

    ts, ds_, jjs = [], [], []
    for j in range(_CHUNK // 16):
        t = base + j * 16 + lax.iota(jnp.int32, 16)
        u = jnp.maximum(t - _HEAD, 0)
        d = lax.div(u, jnp.full((16,), _L + 1, jnp.int32))
        jj = u - d * (_L + 1)
        ts.append(t); ds_.append(d); jjs.append(jj)
        seg_v[pl.ds(j * 16, 16)] = jnp.where(t < _HEAD, 0, d + 1)

    cp_seg = pltpu.async_copy(seg_v, seg_out.at[pl.ds(base, _CHUNK)], sem_s)

    cp_q.wait()
    cp_d.wait()
    for j in range(_CHUNK // 16):
        t, d, jj = ts[j], ds_[j], jjs[j]
        qi = jnp.clip(t - 1, 0, _Q - 1)
        jcl = jnp.minimum(jj, _L - 1)
        qval = plsc.load_gather(q_v, [qi])
        dval = plsc.load_gather(dom_v, [d, jcl])
        val = jnp.where(t < _HEAD, qval,
                        jnp.where(jj == _L, _SEP_ID, dval))
        val = jnp.where(t == 0, _CLS_ID,
                        jnp.where(t == _HEAD - 1, _SEP_ID, val))
        idx_v[pl.ds(j * 16, 16)] = val

    pltpu.async_copy(vocab_hbm.at[idx_v], rows_v, sem_g).wait()

    @pl.when(lax.axis_index("c") == 1)
    def _probe():
        pltpu.async_copy(vocab_hbm.at[idx_v], rows_v, sem_g).wait()
        pltpu.async_copy(vocab_hbm.at[idx_v], rows_v, sem_g).wait()

    pltpu.async_copy(rows_v, ids_out.at[pl.ds(base, _CHUNK)], sem_i).wait()
    cp_seg.wait()


@jax.jit
def kernel(query_tok, domains, vocab_map):
    mesh = plsc.VectorSubcoreMesh(core_axis_name="c", subcore_axis_name="s")
    k = functools.partial(
        pl.kernel,
        out_type=[
            jax.ShapeDtypeStruct((_N,), jnp.int32),
            jax.ShapeDtypeStruct((_N,), jnp.int32),
        ],
        mesh=mesh,
        scratch_types=[
            pltpu.VMEM((_Q,), jnp.int32),
            pltpu.VMEM((_D, _L), jnp.int32),
            pltpu.VMEM((_CHUNK,), jnp.int32),
            pltpu.VMEM((_CHUNK,), jnp.int32),
            pltpu.VMEM((_CHUNK,), jnp.int32),
            pltpu.SemaphoreType.DMA,
            pltpu.SemaphoreType.DMA,
            pltpu.SemaphoreType.DMA,
            pltpu.SemaphoreType.DMA,
            pltpu.SemaphoreType.DMA,
        ],
        compiler_params=pltpu.CompilerParams(needs_layout_passes=False),
    )(_body)
    ids, seg = k(query_tok, domains, vocab_map)
    return ids, seg


# speedup vs baseline: 1.0838x; 1.0838x over previous
import functools

import jax
import jax.numpy as jnp
from jax import lax
from jax.experimental import pallas as pl
from jax.experimental.pallas import tpu as pltpu
from jax.experimental.pallas import tpu_sc as plsc

_VOCAB = 1000000
_CLS_ID = _VOCAB
_SEP_ID = _VOCAB + 1

_Q = 200
_D = 26
_L = 50
_HEAD = _Q + 2
_N = _HEAD + _D * (_L + 1)
_NW = 32
_CHUNK = 48


def _body(query_hbm, domains_hbm, vocab_hbm, ids_out, seg_out,
          q_v, dom_v, idx_v, seg_v, rows_v,
          sem_q, sem_d, sem_g, sem_s, sem_i):
    wid = lax.axis_index("s") * 2 + (1 - lax.axis_index("c"))
    base = pl.multiple_of(jnp.minimum(wid * _CHUNK, _N - _CHUNK), 8)

    needs_q = base < _Q + 1
    needs_d = base + _CHUNK > _HEAD

    @pl.when(needs_q)
    def _():
        pltpu.async_copy(query_hbm, q_v, sem_q)

    @pl.when(needs_d)
    def _():
        pltpu.async_copy(domains_hbm, dom_v, sem_d)

    ts, ds_, jjs = [], [], []
    for j in range(_CHUNK // 16):
        t = base + j * 16 + lax.iota(jnp.int32, 16)
        u = jnp.maximum(t - _HEAD, 0)
        d = lax.div(u, jnp.full((16,), _L + 1, jnp.int32))
        jj = u - d * (_L + 1)
        ts.append(t); ds_.append(d); jjs.append(jj)
        seg_v[pl.ds(j * 16, 16)] = jnp.where(t < _HEAD, 0, d + 1)

    cp_seg = pltpu.async_copy(seg_v, seg_out.at[pl.ds(base, _CHUNK)], sem_s)

    @pl.when(needs_q)
    def _():
        pltpu.make_async_copy(query_hbm, q_v, sem_q).wait()

    @pl.when(needs_d)
    def _():
        pltpu.make_async_copy(domains_hbm, dom_v, sem_d).wait()
    for j in range(_CHUNK // 16):
        t, d, jj = ts[j], ds_[j], jjs[j]
        qi = jnp.clip(t - 1, 0, _Q - 1)
        jcl = jnp.minimum(jj, _L - 1)
        qval = plsc.load_gather(q_v, [qi])
        dval = plsc.load_gather(dom_v, [d, jcl])
        val = jnp.where(t < _HEAD, qval,
                        jnp.where(jj == _L, _SEP_ID, dval))
        val = jnp.where(t == 0, _CLS_ID,
                        jnp.where(t == _HEAD - 1, _SEP_ID, val))
        idx_v[pl.ds(j * 16, 16)] = val

    pltpu.async_copy(vocab_hbm.at[idx_v], rows_v, sem_g).wait()

    pltpu.async_copy(rows_v, ids_out.at[pl.ds(base, _CHUNK)], sem_i).wait()
    cp_seg.wait()


@jax.jit
def kernel(query_tok, domains, vocab_map):
    mesh = plsc.VectorSubcoreMesh(core_axis_name="c", subcore_axis_name="s")
    k = functools.partial(
        pl.kernel,
        out_type=[
            jax.ShapeDtypeStruct((_N,), jnp.int32),
            jax.ShapeDtypeStruct((_N,), jnp.int32),
        ],
        mesh=mesh,
        scratch_types=[
            pltpu.VMEM((_Q,), jnp.int32),
            pltpu.VMEM((_D, _L), jnp.int32),
            pltpu.VMEM((_CHUNK,), jnp.int32),
            pltpu.VMEM((_CHUNK,), jnp.int32),
            pltpu.VMEM((_CHUNK,), jnp.int32),
            pltpu.SemaphoreType.DMA,
            pltpu.SemaphoreType.DMA,
            pltpu.SemaphoreType.DMA,
            pltpu.SemaphoreType.DMA,
            pltpu.SemaphoreType.DMA,
        ],
        compiler_params=pltpu.CompilerParams(needs_layout_passes=False),
    )(_body)
    ids, seg = k(query_tok, domains, vocab_map)
    return ids, seg
